# Initial kernel scaffold; baseline (speedup 1.0000x reference)
import functools

import jax
import jax.numpy as jnp
from jax import lax
from jax.experimental import pallas as pl
from jax.experimental.pallas import tpu as pltpu
from jax.experimental.pallas import tpu_sc as plsc

B, N, K, EMB = 8, 1024, 20, 512
EPS = 1e-5


def _knn_body(ft_ref, idx_ref):
    b = pl.program_id(0)
    f = ft_ref[0]
    inner = lax.dot_general(f, f, (((1,), (1,)), ((), ())),
                            preferred_element_type=jnp.float32)
    lane = lax.broadcasted_iota(jnp.int32, (N, N), 1)
    sub = lax.broadcasted_iota(jnp.int32, (N, N), 0)
    diag = jnp.where(lane == sub, inner, 0.0)
    r_col = jnp.sum(diag, axis=1, keepdims=True)
    r_row = jnp.sum(diag, axis=0, keepdims=True)
    cur = 2.0 * inner - r_col - r_row
    base = b * N
    cols = []
    for _ in range(K):
        m = jnp.max(cur, axis=1, keepdims=True)
        cand = jnp.where(cur == m, lane, N)
        amin = jnp.min(cand, axis=1, keepdims=True)
        cols.append(amin + base)
        cur = jnp.where(lane == amin, -jnp.inf, cur)
    idx_ref[0] = jnp.concatenate(cols, axis=1)


def _knn(ft, interpret=False):
    c = ft.shape[-1]
    return pl.pallas_call(
        _knn_body,
        grid=(B,),
        in_specs=[pl.BlockSpec((1, N, c), lambda b: (b, 0, 0))],
        out_specs=pl.BlockSpec((1, N, K), lambda b: (b, 0, 0)),
        out_shape=jax.ShapeDtypeStruct((B, N, K), jnp.int32),
        interpret=interpret,
    )(ft)


def _gh_body(ft_ref, wg_ref, wh_ref, gt_ref, ht_ref):
    f = ft_ref[0]
    gt_ref[0] = lax.dot_general(f, wg_ref[...], (((1,), (1,)), ((), ())),
                                preferred_element_type=jnp.float32)
    ht_ref[0] = lax.dot_general(f, wh_ref[...], (((1,), (1,)), ((), ())),
                                preferred_element_type=jnp.float32)


def _gh(ft, wg, wh, interpret=False):
    c = ft.shape[-1]
    o = wg.shape[0]
    return pl.pallas_call(
        _gh_body,
        grid=(B,),
        in_specs=[
            pl.BlockSpec((1, N, c), lambda b: (b, 0, 0)),
            pl.BlockSpec((o, c), lambda b: (0, 0)),
            pl.BlockSpec((o, c), lambda b: (0, 0)),
        ],
        out_specs=[
            pl.BlockSpec((1, N, o), lambda b: (b, 0, 0)),
            pl.BlockSpec((1, N, o), lambda b: (b, 0, 0)),
        ],
        out_shape=[
            jax.ShapeDtypeStruct((B, N, o), jnp.float32),
            jax.ShapeDtypeStruct((B, N, o), jnp.float32),
        ],
        interpret=interpret,
    )(ft, wg, wh)


_NW = 32
_P = 4
_RPW = (B * N) // _NW


def _make_sc_gather(o):
    g = _P * K
    mesh = plsc.VectorSubcoreMesh(core_axis_name="c", subcore_axis_name="s")

    @functools.partial(
        pl.kernel,
        out_type=tuple(jax.ShapeDtypeStruct((B * N, o), jnp.float32)
                       for _ in range(4)),
        mesh=mesh,
        scratch_types=[
            pltpu.VMEM((g,), jnp.int32),
            pltpu.VMEM((g, o), jnp.float32),
            pltpu.VMEM((_P, o), jnp.float32),
            pltpu.VMEM((_P, o), jnp.float32),
            pltpu.VMEM((_P, o), jnp.float32),
            pltpu.VMEM((_P, o), jnp.float32),
            pltpu.SemaphoreType.DMA,
        ],
    )
    def sc_gather(gt_hbm, idx_hbm, mx_hbm, mn_hbm, s1_hbm, s2_hbm,
                  idx_v, rows_v, mx_v, mn_v, s1_v, s2_v, sem):
        wid = lax.axis_index("s") * 2 + lax.axis_index("c")
        row0 = wid * _RPW

        def chunk(j, carry):
            r0 = row0 + j * _P
            pltpu.sync_copy(idx_hbm.at[pl.ds(r0 * K, g)], idx_v)
            pltpu.async_copy(gt_hbm.at[idx_v], rows_v, sem).wait()

            def point(p, carry2):
                rbase = p * K

                def colgrp(ci, carry3):
                    c0 = ci * 16
                    v = rows_v[rbase, pl.ds(c0, 16)]
                    mx = v
                    mn = v
                    s1 = v
                    s2 = v * v
                    for k in range(1, K):
                        v = rows_v[rbase + k, pl.ds(c0, 16)]
                        mx = jnp.maximum(mx, v)
                        mn = jnp.minimum(mn, v)
                        s1 = s1 + v
                        s2 = s2 + v * v
                    mx_v[p, pl.ds(c0, 16)] = mx
                    mn_v[p, pl.ds(c0, 16)] = mn
                    s1_v[p, pl.ds(c0, 16)] = s1
                    s2_v[p, pl.ds(c0, 16)] = s2
                    return carry3

                return lax.fori_loop(0, o // 16, colgrp, carry2)

            lax.fori_loop(0, _P, point, 0)
            pltpu.sync_copy(mx_v, mx_hbm.at[pl.ds(r0, _P)])
            pltpu.sync_copy(mn_v, mn_hbm.at[pl.ds(r0, _P)])
            pltpu.sync_copy(s1_v, s1_hbm.at[pl.ds(r0, _P)])
            pltpu.sync_copy(s2_v, s2_hbm.at[pl.ds(r0, _P)])
            return carry

        lax.fori_loop(0, _RPW // _P, chunk, 0)

    return sc_gather


def _statsepi_body(mx_ref, mn_ref, s1_ref, s2_ref, ht_ref, g_ref, b_ref,
                   out_ref):
    s1 = s1_ref[...]
    ht = ht_ref[...]
    cnt = float(B * N * K)
    sum_y = (jnp.sum(s1, axis=0, keepdims=True)
             + float(K) * jnp.sum(ht, axis=0, keepdims=True))
    sum_y2 = (jnp.sum(s2_ref[...], axis=0, keepdims=True)
              + 2.0 * jnp.sum(ht * s1, axis=0, keepdims=True)
              + float(K) * jnp.sum(ht * ht, axis=0, keepdims=True))
    mean = sum_y / cnt
    var = sum_y2 / cnt - mean * mean
    scale = g_ref[...] * lax.rsqrt(var + EPS)
    bias = b_ref[...] - mean * scale
    yext = jnp.where(scale >= 0.0, mx_ref[...], mn_ref[...]) + ht
    out_ref[...] = jnp.maximum(yext * scale + bias, 0.0)


def _statsepi(mx, mn, s1, s2, ht, gamma, beta, interpret=False):
    o = mx.shape[-1]
    return pl.pallas_call(
        _statsepi_body,
        out_shape=jax.ShapeDtypeStruct((B * N, o), jnp.float32),
        interpret=interpret,
    )(mx, mn, s1, s2, ht, gamma.reshape(1, o), beta.reshape(1, o))


def _sb5_body(ft_ref, w_ref, g_ref, b_ref, sb_ref):
    f = ft_ref[...]
    w = w_ref[...]
    smat = lax.dot_general(f, f, (((0,), (0,)), ((), ())),
                           preferred_element_type=jnp.float32)
    mv = jnp.sum(f, axis=0, keepdims=True)
    u = lax.dot_general(w, smat, (((1,), (0,)), ((), ())),
                        preferred_element_type=jnp.float32)
    cnt = float(B * N)
    m2 = jnp.sum(u * w, axis=1, keepdims=True) / cnt
    mean = lax.dot_general(w, mv, (((1,), (1,)), ((), ())),
                           preferred_element_type=jnp.float32) / cnt
    var = m2 - mean * mean
    scale = g_ref[...] * lax.rsqrt(var + EPS)
    bias = b_ref[...] - mean * scale
    sb_ref[...] = jnp.concatenate([scale, bias], axis=1)


def _sb5(ft5, w5, g5, b5, interpret=False):
    return pl.pallas_call(
        _sb5_body,
        out_shape=jax.ShapeDtypeStruct((EMB, 2), jnp.float32),
        interpret=interpret,
    )(ft5, w5, g5.reshape(EMB, 1), b5.reshape(EMB, 1))


def _final_body(ft_ref, w_ref, sb_ref, out_ref):
    f = ft_ref[0]
    y = lax.dot_general(w_ref[...], f, (((1,), (1,)), ((), ())),
                        preferred_element_type=jnp.float32)
    scale = sb_ref[:, 0:1]
    bias = sb_ref[:, 1:2]
    out_ref[0] = jnp.maximum(y * scale + bias, 0.0)


def _final(ft5, w5, sb5, interpret=False):
    c5 = ft5.shape[-1]
    return pl.pallas_call(
        _final_body,
        grid=(B,),
        in_specs=[
            pl.BlockSpec((1, N, c5), lambda b: (b, 0, 0)),
            pl.BlockSpec((EMB, c5), lambda b: (0, 0)),
            pl.BlockSpec((EMB, 2), lambda b: (0, 0)),
        ],
        out_specs=pl.BlockSpec((1, EMB, N), lambda b: (b, 0, 0)),
        out_shape=jax.ShapeDtypeStruct((B, EMB, N), jnp.float32),
        interpret=interpret,
    )(ft5.reshape(B, N, c5), w5, sb5)


def _split_w(w):
    c = w.shape[1] // 2
    wr = w[:, :c]
    return wr, w[:, c:] - wr


def _stage(ft_knn, ft_cat, w, gamma, beta, sc_gather, interpret=False):
    idx = _knn(ft_knn, interpret=interpret).reshape(B * N * K)
    wg, wh = _split_w(w)
    gt, ht = _gh(ft_cat, wg, wh, interpret=interpret)
    o = w.shape[0]
    gt = gt.reshape(B * N, o)
    ht = ht.reshape(B * N, o)
    mx, mn, s1, s2 = sc_gather(gt, idx)
    net = _statsepi(mx, mn, s1, s2, ht, gamma, beta, interpret=interpret)
    return net.reshape(B, N, o)


def kernel(x, W1, W2, W3, W4, W5, g1, b1, g2, b2, g3, b3, g4, b4, g5, b5):
    xt = jnp.transpose(x, (0, 2, 1))
    sc64 = _make_sc_gather(64)
    sc128 = _make_sc_gather(128)

    net1 = _stage(xt, xt, W1, g1, b1, sc64)
    cat2 = jnp.concatenate([xt, net1], axis=-1)
    net2 = _stage(net1, cat2, W2, g2, b2, sc64)
    cat3 = jnp.concatenate([cat2, net2], axis=-1)
    net3 = _stage(net2, cat3, W3, g3, b3, sc64)
    cat4 = jnp.concatenate([cat3, net3], axis=-1)
    net4 = _stage(net3, cat4, W4, g4, b4, sc128)
    cat5 = jnp.concatenate([cat4, net4], axis=-1)

    ft5 = cat5.reshape(B * N, cat5.shape[-1])
    sb5 = _sb5(ft5, W5, g5, b5)
    out = _final(ft5, W5, sb5)
    return (out[:B // 2], out[B // 2:])


# speedup vs baseline: 3.5448x; 3.5448x over previous
import functools

import jax
import jax.numpy as jnp
from jax import lax
from jax.experimental import pallas as pl
from jax.experimental.pallas import tpu as pltpu
from jax.experimental.pallas import tpu_sc as plsc

B, N, K, EMB = 8, 1024, 20, 512
EPS = 1e-5


def _knn_body(ft_ref, xx_ref, idx_ref):
    b = pl.program_id(0)
    f = ft_ref[0]
    inner = lax.dot_general(f, f, (((1,), (1,)), ((), ())),
                            preferred_element_type=jnp.float32)
    lane = lax.broadcasted_iota(jnp.int32, (N, N), 1)
    sub = lax.broadcasted_iota(jnp.int32, (N, N), 0)
    eye = jnp.where(lane == sub, 1.0, 0.0)
    r_row = xx_ref[0]
    r_col = lax.dot_general(eye, r_row, (((1,), (1,)), ((), ())),
                            preferred_element_type=jnp.float32,
                            precision=lax.Precision.HIGHEST)
    cur = -r_row - (-2.0 * inner) - r_col
    base = b * N
    cols = []
    for _ in range(K):
        m = jnp.max(cur, axis=1, keepdims=True)
        cand = jnp.where(cur == m, lane, N)
        amin = jnp.min(cand, axis=1, keepdims=True)
        cols.append(amin + base)
        cur = jnp.where(lane == amin, -jnp.inf, cur)
    idx_ref[0] = jnp.concatenate(cols, axis=1)


def _knn(ft, xx, interpret=False):
    c = ft.shape[-1]
    return pl.pallas_call(
        _knn_body,
        grid=(B,),
        in_specs=[
            pl.BlockSpec((1, N, c), lambda b: (b, 0, 0)),
            pl.BlockSpec((1, 1, N), lambda b: (b, 0, 0)),
        ],
        out_specs=pl.BlockSpec((1, N, K), lambda b: (b, 0, 0)),
        out_shape=jax.ShapeDtypeStruct((B, N, K), jnp.int32),
        interpret=interpret,
    )(ft, xx)


_NW = 32
_RPW = (B * N) // _NW
_P = 8


def _make_sc_rel(cp):
    g2 = (_P * K) // 2
    mesh = plsc.VectorSubcoreMesh(core_axis_name="c", subcore_axis_name="s")

    @functools.partial(
        pl.kernel,
        out_type=jax.ShapeDtypeStruct((B * N * K, cp), jnp.float32),
        mesh=mesh,
        compiler_params=pltpu.CompilerParams(use_tc_tiling_on_sc=False),
        scratch_types=[
            pltpu.VMEM((g2,), jnp.int32),
            pltpu.VMEM((g2,), jnp.int32),
            pltpu.VMEM((_P * K, cp), jnp.float32),
            pltpu.VMEM((_P, cp), jnp.float32),
            pltpu.SemaphoreType.DMA,
        ],
    )
    def sc_rel(ft_hbm, idx_hbm, rel_hbm, idx_a, idx_b, rows_v, fn_v, sem):
        wid = lax.axis_index("s") * 2 + lax.axis_index("c")
        row0 = wid * _RPW

        def chunk(j, carry):
            r0 = row0 + j * _P
            e0 = r0 * K
            pltpu.sync_copy(idx_hbm.at[pl.ds(e0, g2)], idx_a)
            pltpu.sync_copy(idx_hbm.at[pl.ds(e0 + g2, g2)], idx_b)
            c1 = pltpu.make_async_copy(
                ft_hbm.at[idx_a], rows_v.at[pl.ds(0, g2), :], sem)
            c2 = pltpu.make_async_copy(
                ft_hbm.at[idx_b], rows_v.at[pl.ds(g2, g2), :], sem)
            c1.start()
            c2.start()
            pltpu.sync_copy(ft_hbm.at[pl.ds(r0, _P)], fn_v)
            c1.wait()
            c2.wait()

            def point(p, carry2):
                def kgrp(k, carry3):
                    def colgrp(ci, carry4):
                        c0 = ci * 16
                        r = p * K + k
                        rows_v[r, pl.ds(c0, 16)] = (
                            rows_v[r, pl.ds(c0, 16)] - fn_v[p, pl.ds(c0, 16)])
                        return carry4
                    return lax.fori_loop(0, cp // 16, colgrp, carry3)
                return lax.fori_loop(0, K, kgrp, carry2)

            lax.fori_loop(0, _P, point, 0)
            pltpu.sync_copy(rows_v, rel_hbm.at[pl.ds(e0, _P * K)])
            return carry

        lax.fori_loop(0, _RPW // _P, chunk, 0)

    return sc_rel


_NB = 256


def _edgeconv_body(rel_ref, ft_ref, wr_ref, wx_ref,
                   mx_ref, mn_ref, s1_ref, s2_ref, ht_ref):
    wr = wr_ref[...]
    g0 = lax.dot_general(rel_ref[0, :, 0, :], wr, (((1,), (1,)), ((), ())),
                         preferred_element_type=jnp.float32)
    mx = g0
    mn = g0
    s1 = g0
    s2 = g0 * g0
    for k in range(1, K):
        gk = lax.dot_general(rel_ref[0, :, k, :], wr,
                             (((1,), (1,)), ((), ())),
                             preferred_element_type=jnp.float32)
        mx = jnp.maximum(mx, gk)
        mn = jnp.minimum(mn, gk)
        s1 = s1 + gk
        s2 = s2 + gk * gk
    mx_ref[0] = mx
    mn_ref[0] = mn
    s1_ref[0] = s1
    s2_ref[0] = s2
    ht_ref[0] = lax.dot_general(ft_ref[0], wx_ref[...],
                                (((1,), (1,)), ((), ())),
                                preferred_element_type=jnp.float32)


def _edgeconv(rel, ft, wr, wx, interpret=False):
    cp = ft.shape[-1]
    o = wr.shape[0]
    out = functools.partial(
        pl.BlockSpec, (1, _NB, o), lambda b, i: (b, i, 0))
    res = pl.pallas_call(
        _edgeconv_body,
        grid=(B, N // _NB),
        in_specs=[
            pl.BlockSpec((1, _NB, K, cp), lambda b, i: (b, i, 0, 0)),
            pl.BlockSpec((1, _NB, cp), lambda b, i: (b, i, 0)),
            pl.BlockSpec((o, cp), lambda b, i: (0, 0)),
            pl.BlockSpec((o, cp), lambda b, i: (0, 0)),
        ],
        out_specs=[out() for _ in range(5)],
        out_shape=[jax.ShapeDtypeStruct((B, N, o), jnp.float32)
                   for _ in range(5)],
        interpret=interpret,
    )(rel.reshape(B, N, K, cp), ft, wr, wx)
    return [r.reshape(B * N, o) for r in res]


def _gh_body(ft_ref, wg_ref, wh_ref, gt_ref, ht_ref):
    f = ft_ref[0]
    gt_ref[0] = lax.dot_general(f, wg_ref[...], (((1,), (1,)), ((), ())),
                                preferred_element_type=jnp.float32)
    ht_ref[0] = lax.dot_general(f, wh_ref[...], (((1,), (1,)), ((), ())),
                                preferred_element_type=jnp.float32)


def _gh(ft, wg, wh, interpret=False):
    c = ft.shape[-1]
    o = wg.shape[0]
    return pl.pallas_call(
        _gh_body,
        grid=(B,),
        in_specs=[
            pl.BlockSpec((1, N, c), lambda b: (b, 0, 0)),
            pl.BlockSpec((o, c), lambda b: (0, 0)),
            pl.BlockSpec((o, c), lambda b: (0, 0)),
        ],
        out_specs=[
            pl.BlockSpec((1, N, o), lambda b: (b, 0, 0)),
            pl.BlockSpec((1, N, o), lambda b: (b, 0, 0)),
        ],
        out_shape=[
            jax.ShapeDtypeStruct((B, N, o), jnp.float32),
            jax.ShapeDtypeStruct((B, N, o), jnp.float32),
        ],
        interpret=interpret,
    )(ft, wg, wh)


_P4 = 4


def _make_sc_gather(o):
    g = _P4 * K
    mesh = plsc.VectorSubcoreMesh(core_axis_name="c", subcore_axis_name="s")

    @functools.partial(
        pl.kernel,
        out_type=tuple(jax.ShapeDtypeStruct((B * N, o), jnp.float32)
                       for _ in range(4)),
        mesh=mesh,
        compiler_params=pltpu.CompilerParams(use_tc_tiling_on_sc=False),
        scratch_types=[
            pltpu.VMEM((g,), jnp.int32),
            pltpu.VMEM((g, o), jnp.float32),
            pltpu.VMEM((_P4, o), jnp.float32),
            pltpu.VMEM((_P4, o), jnp.float32),
            pltpu.VMEM((_P4, o), jnp.float32),
            pltpu.VMEM((_P4, o), jnp.float32),
            pltpu.SemaphoreType.DMA,
        ],
    )
    def sc_gather(gt_hbm, idx_hbm, mx_hbm, mn_hbm, s1_hbm, s2_hbm,
                  idx_v, rows_v, mx_v, mn_v, s1_v, s2_v, sem):
        wid = lax.axis_index("s") * 2 + lax.axis_index("c")
        row0 = wid * _RPW

        def chunk(j, carry):
            r0 = row0 + j * _P4
            pltpu.sync_copy(idx_hbm.at[pl.ds(r0 * K, g)], idx_v)
            pltpu.async_copy(gt_hbm.at[idx_v], rows_v, sem).wait()

            def point(p, carry2):
                rbase = p * K

                def colgrp(ci, carry3):
                    c0 = ci * 16
                    v = rows_v[rbase, pl.ds(c0, 16)]
                    mx = v
                    mn = v
                    s1 = v
                    s2 = v * v
                    for k in range(1, K):
                        v = rows_v[rbase + k, pl.ds(c0, 16)]
                        mx = jnp.maximum(mx, v)
                        mn = jnp.minimum(mn, v)
                        s1 = s1 + v
                        s2 = s2 + v * v
                    mx_v[p, pl.ds(c0, 16)] = mx
                    mn_v[p, pl.ds(c0, 16)] = mn
                    s1_v[p, pl.ds(c0, 16)] = s1
                    s2_v[p, pl.ds(c0, 16)] = s2
                    return carry3

                return lax.fori_loop(0, o // 16, colgrp, carry2)

            lax.fori_loop(0, _P4, point, 0)
            pltpu.sync_copy(mx_v, mx_hbm.at[pl.ds(r0, _P4)])
            pltpu.sync_copy(mn_v, mn_hbm.at[pl.ds(r0, _P4)])
            pltpu.sync_copy(s1_v, s1_hbm.at[pl.ds(r0, _P4)])
            pltpu.sync_copy(s2_v, s2_hbm.at[pl.ds(r0, _P4)])
            return carry

        lax.fori_loop(0, _RPW // _P4, chunk, 0)

    return sc_gather


def _statsepi_body(mx_ref, mn_ref, s1_ref, s2_ref, ht_ref, g_ref, b_ref,
                   out_ref):
    s1 = s1_ref[...]
    ht = ht_ref[...]
    cnt = float(B * N * K)
    sum_y = (jnp.sum(s1, axis=0, keepdims=True)
             + float(K) * jnp.sum(ht, axis=0, keepdims=True))
    sum_y2 = (jnp.sum(s2_ref[...], axis=0, keepdims=True)
              + 2.0 * jnp.sum(ht * s1, axis=0, keepdims=True)
              + float(K) * jnp.sum(ht * ht, axis=0, keepdims=True))
    mean = sum_y / cnt
    var = sum_y2 / cnt - mean * mean
    scale = g_ref[...] * lax.rsqrt(var + EPS)
    bias = b_ref[...] - mean * scale
    yext = jnp.where(scale >= 0.0, mx_ref[...], mn_ref[...]) + ht
    out_ref[...] = jnp.maximum(yext * scale + bias, 0.0)


def _statsepi(mx, mn, s1, s2, ht, gamma, beta, interpret=False):
    o = mx.shape[-1]
    return pl.pallas_call(
        _statsepi_body,
        out_shape=jax.ShapeDtypeStruct((B * N, o), jnp.float32),
        interpret=interpret,
    )(mx, mn, s1, s2, ht, gamma.reshape(1, o), beta.reshape(1, o))


def _sb5_body(ft_ref, w_ref, g_ref, b_ref, sb_ref):
    f = ft_ref[...]
    w = w_ref[...]
    smat = lax.dot_general(f, f, (((0,), (0,)), ((), ())),
                           preferred_element_type=jnp.float32,
                           precision=lax.Precision.HIGHEST)
    mv = jnp.sum(f, axis=0, keepdims=True)
    u = lax.dot_general(w, smat, (((1,), (0,)), ((), ())),
                        preferred_element_type=jnp.float32,
                        precision=lax.Precision.HIGHEST)
    cnt = float(B * N)
    m2 = jnp.sum(u * w, axis=1, keepdims=True) / cnt
    mean = lax.dot_general(w, mv, (((1,), (1,)), ((), ())),
                           preferred_element_type=jnp.float32,
                           precision=lax.Precision.HIGHEST) / cnt
    var = m2 - mean * mean
    scale = g_ref[...] * lax.rsqrt(var + EPS)
    bias = b_ref[...] - mean * scale
    sb_ref[...] = jnp.concatenate([scale, bias], axis=1)


def _sb5(ft5, w5, g5, b5, interpret=False):
    return pl.pallas_call(
        _sb5_body,
        out_shape=jax.ShapeDtypeStruct((EMB, 2), jnp.float32),
        interpret=interpret,
    )(ft5, w5, g5.reshape(EMB, 1), b5.reshape(EMB, 1))


def _final_body(ft_ref, w_ref, sb_ref, out_ref):
    f = ft_ref[0]
    y = lax.dot_general(w_ref[...], f, (((1,), (1,)), ((), ())),
                        preferred_element_type=jnp.float32)
    scale = sb_ref[:, 0:1]
    bias = sb_ref[:, 1:2]
    out_ref[0] = jnp.maximum(y * scale + bias, 0.0)


def _final(ft5, w5, sb5, interpret=False):
    c5 = ft5.shape[-1]
    return pl.pallas_call(
        _final_body,
        grid=(B,),
        in_specs=[
            pl.BlockSpec((1, N, c5), lambda b: (b, 0, 0)),
            pl.BlockSpec((EMB, c5), lambda b: (0, 0)),
            pl.BlockSpec((EMB, 2), lambda b: (0, 0)),
        ],
        out_specs=pl.BlockSpec((1, EMB, N), lambda b: (b, 0, 0)),
        out_shape=jax.ShapeDtypeStruct((B, EMB, N), jnp.float32),
        interpret=interpret,
    )(ft5.reshape(B, N, c5), w5, sb5)


def _pad_lanes(a, cp):
    c = a.shape[-1]
    if c == cp:
        return a
    return jnp.pad(a, [(0, 0)] * (a.ndim - 1) + [(0, cp - c)])


def _xx_like_ref(ft):
    fc = jnp.transpose(ft, (0, 2, 1))
    return jnp.sum(fc * fc, axis=1, keepdims=True)


def _stage123(ft_knn, ft_cat, w, gamma, beta, sc_rel, cp, interpret=False):
    idx = _knn(ft_knn, _xx_like_ref(ft_knn),
               interpret=interpret).reshape(B * N * K)
    o = w.shape[0]
    c = ft_cat.shape[-1]
    ftp = _pad_lanes(ft_cat, cp)
    wr = _pad_lanes(w[:, :c], cp)
    wx = _pad_lanes(w[:, c:], cp)
    rel = sc_rel(ftp.reshape(B * N, cp), idx)
    mx, mn, s1, s2, ht = _edgeconv(rel, ftp, wr, wx, interpret=interpret)
    net = _statsepi(mx, mn, s1, s2, ht, gamma, beta, interpret=interpret)
    return net.reshape(B, N, o)


def _stage4(ft_knn, ft_cat, w, gamma, beta, sc_gather, interpret=False):
    idx = _knn(ft_knn, _xx_like_ref(ft_knn),
               interpret=interpret).reshape(B * N * K)
    c = ft_cat.shape[-1]
    wg = w[:, :c]
    wh = w[:, c:] - wg
    gt, ht = _gh(ft_cat, wg, wh, interpret=interpret)
    o = w.shape[0]
    gt = gt.reshape(B * N, o)
    ht = ht.reshape(B * N, o)
    mx, mn, s1, s2 = sc_gather(gt, idx)
    net = _statsepi(mx, mn, s1, s2, ht, gamma, beta, interpret=interpret)
    return net.reshape(B, N, o)


def kernel(x, W1, W2, W3, W4, W5, g1, b1, g2, b2, g3, b3, g4, b4, g5, b5):
    xt = jnp.transpose(x, (0, 2, 1))

    net1 = _stage123(xt, xt, W1, g1, b1, _make_sc_rel(16), 16)
    cat2 = jnp.concatenate([xt, net1], axis=-1)
    net2 = _stage123(net1, cat2, W2, g2, b2, _make_sc_rel(80), 80)
    cat3 = jnp.concatenate([cat2, net2], axis=-1)
    net3 = _stage123(net2, cat3, W3, g3, b3, _make_sc_rel(144), 144)
    cat4 = jnp.concatenate([cat3, net3], axis=-1)
    net4 = _stage4(net3, cat4, W4, g4, b4, _make_sc_gather(128))
    cat5 = jnp.concatenate([cat4, net4], axis=-1)

    ft5 = cat5.reshape(B * N, cat5.shape[-1])
    sb5 = _sb5(ft5, W5, g5, b5)
    out = _final(ft5, W5, sb5)
    return (out[:B // 2], out[B // 2:])
